# baseline (speedup 1.0000x reference)
import jax
import jax.numpy as jnp
from jax import lax
from jax.experimental import pallas as pl
from jax.experimental.pallas import tpu as pltpu
from jax.experimental.pallas import tpu_sc as plsc

_D = 2048
_K = 36
_KP = 48
_NV = _KP // 16
_JITTER = 0.02
_NC = 2
_NS = 16
_NW = _NC * _NS
_L = 16
_CHUNK = 8
_NB = 6
_DIST = 3
_SB = 128


def _sc_body(x_hbm, n_hbm, idx_hbm, perm_hbm, gate_hbm,
             out_hbm, g_hbm,
             xbufs, nbufs, statsbuf, idxv, permv, meanv, gbuf, stage48,
             allv, sbuf, xin_sems, nin_sems, out_sems):
    rows = x_hbm.shape[0]
    rpw = rows // _NW
    cid = lax.axis_index("c")
    sid = lax.axis_index("s")
    wid = sid * _NC + cid
    base = wid * rpw

    lanes = lax.iota(jnp.int32, _L)
    zeros_i = jnp.zeros((_L,), jnp.int32)
    zeros_f = jnp.zeros((_L,), jnp.float32)
    masks = [lanes < (_K - _L * v) for v in range(_NV)]

    def _prologue_in(b, ci):
        r0 = base + ci * _CHUNK
        pltpu.async_copy(x_hbm.at[pl.ds(r0, _CHUNK)], xbufs[b],
                         xin_sems[b])
        pltpu.async_copy(n_hbm.at[pl.ds(r0, _CHUNK)], nbufs[b],
                         nin_sems[b])

    for v in range(_NV):
        idxv[pl.ds(_L * v, _L)] = zeros_i
        permv[pl.ds(_L * v, _L)] = zeros_i
    gbuf[...] = zeros_f
    pltpu.sync_copy(idx_hbm, idxv.at[pl.ds(0, _K)])
    pltpu.sync_copy(perm_hbm, permv.at[pl.ds(0, _K)])
    pltpu.sync_copy(gate_hbm, gbuf.at[pl.ds(0, 1)])

    gv = plsc.load_gather(gbuf, [zeros_i])
    g_sig = 1.0 / (1.0 + jnp.exp(-gv))
    sv_r = g_sig * _JITTER

    for b in range(_DIST):
        _prologue_in(b, b)

    acc = [zeros_f for _ in range(_NV)]
    srow0 = sid * (rows // _NS)

    def stats_chunk(k, accs):
        pltpu.sync_copy(n_hbm.at[pl.ds(srow0 + k * _SB, _SB)], statsbuf)

        def stats_row(r, a):
            rv = jnp.full((_L,), r, jnp.int32)
            tail = jnp.where(
                masks[2],
                plsc.load_gather(statsbuf, [rv, lanes + 2 * _L],
                                 mask=masks[2]),
                0.0)
            return (a[0] + statsbuf[r, pl.ds(0, _L)],
                    a[1] + statsbuf[r, pl.ds(_L, _L)],
                    a[2] + tail)

        return lax.fori_loop(0, _SB, stats_row, accs)

    acc = lax.fori_loop(0, rows // _NS // _SB, stats_chunk, tuple(acc))
    for v in range(_NV):
        stage48[pl.ds(_L * v, _L)] = acc[v]
    pltpu.sync_copy(stage48, sbuf.at[pl.ds(sid * _KP, _KP)])
    plsc.subcore_barrier()
    pltpu.sync_copy(sbuf, allv)
    scale = _JITTER / rows
    for v in range(_NV):
        tot = zeros_f
        for t in range(_NS):
            tot = tot + allv[pl.ds(t * _KP + _L * v, _L)]
        meanv[pl.ds(_L * v, _L)] = tot * scale

    nsrc_r = [permv[pl.ds(_L * v, _L)] for v in range(_NV)]
    dst_r = [idxv[pl.ds(_L * v, _L)] for v in range(_NV)]
    src_r = [plsc.load_gather(idxv, [nsrc_r[v]]) for v in range(_NV)]
    off_r = [g_sig * plsc.load_gather(meanv, [nsrc_r[v]])
             for v in range(_NV)]

    @pl.when(wid == 0)
    def _():
        gbuf[...] = g_sig
        pltpu.sync_copy(gbuf, g_hbm)

    nchunks = rpw // _CHUNK

    start_in = _prologue_in

    def wait_in(b):
        pltpu.make_async_copy(x_hbm.at[pl.ds(0, _CHUNK)], xbufs[b],
                              xin_sems[b]).wait()
        pltpu.make_async_copy(n_hbm.at[pl.ds(0, _CHUNK)], nbufs[b],
                              nin_sems[b]).wait()

    def start_out(b, ci):
        r0 = base + ci * _CHUNK
        pltpu.async_copy(xbufs[b], out_hbm.at[pl.ds(r0, _CHUNK)],
                         out_sems[b])

    def wait_out(b):
        pltpu.make_async_copy(xbufs[b], out_hbm.at[pl.ds(0, _CHUNK)],
                              out_sems[b]).wait()

    def compute(b):
        def row_body(r, rcarry):
            rv = jnp.full((_L,), r, jnp.int32)
            vals = []
            for v in range(_NV):
                xg = plsc.load_gather(xbufs[b], [rv, src_r[v]],
                                      mask=masks[v])
                ng = plsc.load_gather(nbufs[b], [rv, nsrc_r[v]],
                                      mask=masks[v])
                vals.append(xg + sv_r * ng - off_r[v])
            for v in range(_NV):
                plsc.store_scatter(xbufs[b], [rv, dst_r[v]], vals[v],
                                   mask=masks[v])
            return rcarry

        lax.fori_loop(0, _CHUNK, row_body, 0)

    def phase(b, ci):
        wait_in(b)
        compute(b)
        start_out(b, ci)
        b2 = (b + _DIST) % _NB

        @pl.when(jnp.asarray(ci >= _DIST))
        def _():
            wait_out(b2)

        @pl.when(jnp.asarray(ci + _DIST < nchunks))
        def _():
            start_in(b2, ci + _DIST)

    def group_body(gi, carry):
        for b in range(_NB):
            phase(b, gi * _NB + b)
        return carry

    groups = nchunks // _NB
    lax.fori_loop(0, groups, group_body, 0)
    for t in range(nchunks % _NB):
        phase(t, groups * _NB + t)
    for t in range(_DIST):
        wait_out((nchunks - _DIST + t) % _NB)


def kernel(x, gate, noise, idx, perm):
    b, t, d = x.shape
    rows = b * t
    x2 = x.reshape(rows, d)
    n2 = noise.reshape(rows, _K)

    mesh = plsc.VectorSubcoreMesh(core_axis_name="c", subcore_axis_name="s",
                                  num_cores=_NC, num_subcores=_NS)
    out2, gout = pl.kernel(
        _sc_body,
        out_type=[
            jax.ShapeDtypeStruct((rows, d), jnp.float32),
            jax.ShapeDtypeStruct((_L,), jnp.float32),
        ],
        mesh=mesh,
        compiler_params=pltpu.CompilerParams(needs_layout_passes=False),
        scratch_types=[
            [pltpu.VMEM((_CHUNK, d), jnp.float32) for _ in range(_NB)],
            [pltpu.VMEM((_CHUNK, _K), jnp.float32) for _ in range(_NB)],
            pltpu.VMEM((_SB, _K), jnp.float32),
            pltpu.VMEM((_KP,), jnp.int32),
            pltpu.VMEM((_KP,), jnp.int32),
            pltpu.VMEM((_KP,), jnp.float32),
            pltpu.VMEM((_L,), jnp.float32),
            pltpu.VMEM((_KP,), jnp.float32),
            pltpu.VMEM((_NS * _KP,), jnp.float32),
            pltpu.VMEM_SHARED((_NS * _KP,), jnp.float32),
            [pltpu.SemaphoreType.DMA for _ in range(_NB)],
            [pltpu.SemaphoreType.DMA for _ in range(_NB)],
            [pltpu.SemaphoreType.DMA for _ in range(_NB)],
        ],
    )(x2, n2, idx.astype(jnp.int32), perm.astype(jnp.int32),
      gate.reshape(1).astype(jnp.float32))

    return out2.reshape(b, t, d), gout[0]


# speedup vs baseline: 1.0309x; 1.0309x over previous
import jax
import jax.numpy as jnp
from jax import lax
from jax.experimental import pallas as pl
from jax.experimental.pallas import tpu as pltpu
from jax.experimental.pallas import tpu_sc as plsc

_D = 2048
_K = 36
_KP = 48
_NV = _KP // 16
_JITTER = 0.02
_NC = 2
_NS = 16
_NW = _NC * _NS
_L = 16
_CHUNK = 8
_NB = 4
_SB = 128


def _sc_body(x_hbm, n_hbm, idx_hbm, perm_hbm, gate_hbm,
             out_hbm, g_hbm,
             xbufs, nbufs, statsbufs, stat_sems, idxv, permv, meanv, gbuf,
             stage48,
             allv, sbuf, xin_sems, nin_sems, out_sems):
    rows = x_hbm.shape[0]
    rpw = rows // _NW
    cid = lax.axis_index("c")
    sid = lax.axis_index("s")
    wid = sid * _NC + cid
    base = wid * rpw

    lanes = lax.iota(jnp.int32, _L)
    zeros_i = jnp.zeros((_L,), jnp.int32)
    zeros_f = jnp.zeros((_L,), jnp.float32)
    masks = [lanes < (_K - _L * v) for v in range(_NV)]

    def _prologue_in(b, ci):
        r0 = base + ci * _CHUNK
        pltpu.async_copy(x_hbm.at[pl.ds(r0, _CHUNK)], xbufs[b],
                         xin_sems[b])
        pltpu.async_copy(n_hbm.at[pl.ds(r0, _CHUNK)], nbufs[b],
                         nin_sems[b])

    for v in range(_NV):
        idxv[pl.ds(_L * v, _L)] = zeros_i
        permv[pl.ds(_L * v, _L)] = zeros_i
    gbuf[...] = zeros_f
    pltpu.sync_copy(idx_hbm, idxv.at[pl.ds(0, _K)])
    pltpu.sync_copy(perm_hbm, permv.at[pl.ds(0, _K)])
    pltpu.sync_copy(gate_hbm, gbuf.at[pl.ds(0, 1)])

    gv = plsc.load_gather(gbuf, [zeros_i])
    g_sig = 1.0 / (1.0 + jnp.exp(-gv))
    sv_r = g_sig * _JITTER

    _prologue_in(0, 0)
    _prologue_in(1, 1)

    acc = [zeros_f for _ in range(_NV)]
    srow0 = sid * (rows // _NS)
    nstat = rows // _NS // _SB

    def stats_start(sb, k):
        pltpu.async_copy(n_hbm.at[pl.ds(srow0 + k * _SB, _SB)],
                         statsbufs[sb], stat_sems[sb])

    def stats_wait(sb):
        pltpu.make_async_copy(n_hbm.at[pl.ds(0, _SB)], statsbufs[sb],
                              stat_sems[sb]).wait()

    stats_start(0, 0)
    stats_start(1, 1)

    def stats_one(sb, k, accs):
        stats_wait(sb)

        def stats_row(r2, a):
            for rr in range(2):
                r = r2 * 2 + rr
                rv = jnp.full((_L,), r, jnp.int32)
                tail = jnp.where(
                    masks[2],
                    plsc.load_gather(statsbufs[sb], [rv, lanes + 2 * _L],
                                     mask=masks[2]),
                    0.0)
                a = (a[0] + statsbufs[sb][r, pl.ds(0, _L)],
                     a[1] + statsbufs[sb][r, pl.ds(_L, _L)],
                     a[2] + tail)
            return a

        accs = lax.fori_loop(0, _SB // 2, stats_row, accs)

        @pl.when(k + 2 < nstat)
        def _():
            stats_start(sb, k + 2)

        return accs

    def stats_pair(kp, accs):
        accs = stats_one(0, 2 * kp, accs)
        accs = stats_one(1, 2 * kp + 1, accs)
        return accs

    acc = lax.fori_loop(0, nstat // 2, stats_pair, tuple(acc))
    for v in range(_NV):
        stage48[pl.ds(_L * v, _L)] = acc[v]
    pltpu.sync_copy(stage48, sbuf.at[pl.ds(sid * _KP, _KP)])
    plsc.subcore_barrier()
    pltpu.sync_copy(sbuf, allv)
    scale = _JITTER / rows
    for v in range(_NV):
        tot = zeros_f
        for t in range(_NS):
            tot = tot + allv[pl.ds(t * _KP + _L * v, _L)]
        meanv[pl.ds(_L * v, _L)] = tot * scale

    nsrc_r = [permv[pl.ds(_L * v, _L)] for v in range(_NV)]
    dst_r = [idxv[pl.ds(_L * v, _L)] for v in range(_NV)]
    src_r = [plsc.load_gather(idxv, [nsrc_r[v]]) for v in range(_NV)]
    off_r = [g_sig * plsc.load_gather(meanv, [nsrc_r[v]])
             for v in range(_NV)]

    @pl.when(wid == 0)
    def _():
        gbuf[...] = g_sig
        pltpu.sync_copy(gbuf, g_hbm)

    nchunks = rpw // _CHUNK

    start_in = _prologue_in

    def wait_in(b):
        pltpu.make_async_copy(x_hbm.at[pl.ds(0, _CHUNK)], xbufs[b],
                              xin_sems[b]).wait()
        pltpu.make_async_copy(n_hbm.at[pl.ds(0, _CHUNK)], nbufs[b],
                              nin_sems[b]).wait()

    def start_out(b, ci):
        r0 = base + ci * _CHUNK
        pltpu.async_copy(xbufs[b], out_hbm.at[pl.ds(r0, _CHUNK)],
                         out_sems[b])

    def wait_out(b):
        pltpu.make_async_copy(xbufs[b], out_hbm.at[pl.ds(0, _CHUNK)],
                              out_sems[b]).wait()

    def compute(b):
        def row_body(r, rcarry):
            rv = jnp.full((_L,), r, jnp.int32)
            vals = []
            for v in range(_NV):
                xg = plsc.load_gather(xbufs[b], [rv, src_r[v]],
                                      mask=masks[v])
                ng = plsc.load_gather(nbufs[b], [rv, nsrc_r[v]],
                                      mask=masks[v])
                vals.append(xg + sv_r * ng - off_r[v])
            for v in range(_NV):
                plsc.store_scatter(xbufs[b], [rv, dst_r[v]], vals[v],
                                   mask=masks[v])
            return rcarry

        lax.fori_loop(0, _CHUNK, row_body, 0)

    def group_body(gi, carry):
        for b in range(_NB):
            ci = gi * _NB + b
            wait_in(b)
            compute(b)
            start_out(b, ci)
            b2 = (b + 2) % _NB

            @pl.when(ci >= 2)
            def _():
                wait_out(b2)

            @pl.when(ci + 2 < nchunks)
            def _():
                start_in(b2, ci + 2)

        return carry

    lax.fori_loop(0, nchunks // _NB, group_body, 0)
    wait_out((nchunks - 2) % _NB)
    wait_out((nchunks - 1) % _NB)


def kernel(x, gate, noise, idx, perm):
    b, t, d = x.shape
    rows = b * t
    x2 = x.reshape(rows, d)
    n2 = noise.reshape(rows, _K)

    mesh = plsc.VectorSubcoreMesh(core_axis_name="c", subcore_axis_name="s",
                                  num_cores=_NC, num_subcores=_NS)
    out2, gout = pl.kernel(
        _sc_body,
        out_type=[
            jax.ShapeDtypeStruct((rows, d), jnp.float32),
            jax.ShapeDtypeStruct((_L,), jnp.float32),
        ],
        mesh=mesh,
        compiler_params=pltpu.CompilerParams(needs_layout_passes=False),
        scratch_types=[
            [pltpu.VMEM((_CHUNK, d), jnp.float32) for _ in range(_NB)],
            [pltpu.VMEM((_CHUNK, _K), jnp.float32) for _ in range(_NB)],
            [pltpu.VMEM((_SB, _K), jnp.float32) for _ in range(2)],
            [pltpu.SemaphoreType.DMA for _ in range(2)],
            pltpu.VMEM((_KP,), jnp.int32),
            pltpu.VMEM((_KP,), jnp.int32),
            pltpu.VMEM((_KP,), jnp.float32),
            pltpu.VMEM((_L,), jnp.float32),
            pltpu.VMEM((_KP,), jnp.float32),
            pltpu.VMEM((_NS * _KP,), jnp.float32),
            pltpu.VMEM_SHARED((_NS * _KP,), jnp.float32),
            [pltpu.SemaphoreType.DMA for _ in range(_NB)],
            [pltpu.SemaphoreType.DMA for _ in range(_NB)],
            [pltpu.SemaphoreType.DMA for _ in range(_NB)],
        ],
    )(x2, n2, idx.astype(jnp.int32), perm.astype(jnp.int32),
      gate.reshape(1).astype(jnp.float32))

    return out2.reshape(b, t, d), gout[0]


# speedup vs baseline: 1.0313x; 1.0003x over previous
import jax
import jax.numpy as jnp
from jax import lax
from jax.experimental import pallas as pl
from jax.experimental.pallas import tpu as pltpu
from jax.experimental.pallas import tpu_sc as plsc

_D = 2048
_K = 36
_KP = 48
_NV = _KP // 16
_JITTER = 0.02
_NC = 2
_NS = 16
_NW = _NC * _NS
_L = 16
_CHUNK = 8
_NB = 4
_SB = 128


def _sc_body(x_hbm, n_hbm, idx_hbm, perm_hbm, gate_hbm,
             out_hbm, g_hbm,
             xbufs, nbufs, statsbufs, stat_sems, idxv, permv, meanv, gbuf,
             stage48,
             allv, sbuf, xin_sems, nin_sems, out_sems):
    rows = x_hbm.shape[0]
    rpw = rows // _NW
    cid = lax.axis_index("c")
    sid = lax.axis_index("s")
    wid = sid * _NC + cid
    base = wid * rpw

    lanes = lax.iota(jnp.int32, _L)
    zeros_i = jnp.zeros((_L,), jnp.int32)
    zeros_f = jnp.zeros((_L,), jnp.float32)
    masks = [lanes < (_K - _L * v) for v in range(_NV)]

    def _prologue_in(b, ci):
        r0 = base + ci * _CHUNK
        pltpu.async_copy(x_hbm.at[pl.ds(r0, _CHUNK)], xbufs[b],
                         xin_sems[b])
        pltpu.async_copy(n_hbm.at[pl.ds(r0, _CHUNK)], nbufs[b],
                         nin_sems[b])

    for v in range(_NV):
        idxv[pl.ds(_L * v, _L)] = zeros_i
        permv[pl.ds(_L * v, _L)] = zeros_i
    gbuf[...] = zeros_f
    pltpu.sync_copy(idx_hbm, idxv.at[pl.ds(0, _K)])
    pltpu.sync_copy(perm_hbm, permv.at[pl.ds(0, _K)])
    pltpu.sync_copy(gate_hbm, gbuf.at[pl.ds(0, 1)])

    gv = plsc.load_gather(gbuf, [zeros_i])
    g_sig = 1.0 / (1.0 + jnp.exp(-gv))
    sv_r = g_sig * _JITTER

    _prologue_in(0, 0)
    _prologue_in(1, 1)

    acc = [zeros_f for _ in range(_NV)]
    srow0 = sid * (rows // _NS)
    nstat = rows // _NS // _SB

    def stats_start(sb, k):
        pltpu.async_copy(n_hbm.at[pl.ds(srow0 + k * _SB, _SB)],
                         statsbufs[sb], stat_sems[sb])

    def stats_wait(sb):
        pltpu.make_async_copy(n_hbm.at[pl.ds(0, _SB)], statsbufs[sb],
                              stat_sems[sb]).wait()

    stats_start(0, 0)
    stats_start(1, 1)

    def stats_one(sb, k, accs):
        stats_wait(sb)

        def stats_row(r2, a):
            for rr in range(2):
                r = r2 * 2 + rr
                rv = jnp.full((_L,), r, jnp.int32)
                tail = jnp.where(
                    masks[2],
                    plsc.load_gather(statsbufs[sb], [rv, lanes + 2 * _L],
                                     mask=masks[2]),
                    0.0)
                a = (a[0] + statsbufs[sb][r, pl.ds(0, _L)],
                     a[1] + statsbufs[sb][r, pl.ds(_L, _L)],
                     a[2] + tail)
            return a

        accs = lax.fori_loop(0, _SB // 2, stats_row, accs)

        @pl.when(k + 2 < nstat)
        def _():
            stats_start(sb, k + 2)

        return accs

    def stats_pair(kp, accs):
        accs = stats_one(0, 2 * kp, accs)
        accs = stats_one(1, 2 * kp + 1, accs)
        return accs

    acc = lax.fori_loop(0, nstat // 2, stats_pair, tuple(acc))
    for v in range(_NV):
        stage48[pl.ds(_L * v, _L)] = acc[v]
    pltpu.sync_copy(stage48, sbuf.at[pl.ds(sid * _KP, _KP)])
    plsc.subcore_barrier()
    pltpu.sync_copy(sbuf, allv)
    scale = _JITTER / rows
    for v in range(_NV):
        tot = zeros_f
        for t in range(_NS):
            tot = tot + allv[pl.ds(t * _KP + _L * v, _L)]
        meanv[pl.ds(_L * v, _L)] = tot * scale

    nsrc_r = [permv[pl.ds(_L * v, _L)] for v in range(_NV)]
    dst_r = [idxv[pl.ds(_L * v, _L)] for v in range(_NV)]
    src_r = [plsc.load_gather(idxv, [nsrc_r[v]]) for v in range(_NV)]
    off_r = [g_sig * plsc.load_gather(meanv, [nsrc_r[v]])
             for v in range(_NV)]

    nchunks = rpw // _CHUNK

    start_in = _prologue_in

    def wait_in(b):
        pltpu.make_async_copy(x_hbm.at[pl.ds(0, _CHUNK)], xbufs[b],
                              xin_sems[b]).wait()
        pltpu.make_async_copy(n_hbm.at[pl.ds(0, _CHUNK)], nbufs[b],
                              nin_sems[b]).wait()

    def start_out(b, ci):
        r0 = base + ci * _CHUNK
        pltpu.async_copy(xbufs[b], out_hbm.at[pl.ds(r0, _CHUNK)],
                         out_sems[b])

    def wait_out(b):
        pltpu.make_async_copy(xbufs[b], out_hbm.at[pl.ds(0, _CHUNK)],
                              out_sems[b]).wait()

    def compute(b):
        def row_body(r, rcarry):
            rv = jnp.full((_L,), r, jnp.int32)
            vals = []
            for v in range(_NV):
                xg = plsc.load_gather(xbufs[b], [rv, src_r[v]],
                                      mask=masks[v])
                ng = plsc.load_gather(nbufs[b], [rv, nsrc_r[v]],
                                      mask=masks[v])
                vals.append(xg + sv_r * ng - off_r[v])
            for v in range(_NV):
                plsc.store_scatter(xbufs[b], [rv, dst_r[v]], vals[v],
                                   mask=masks[v])
            return rcarry

        lax.fori_loop(0, _CHUNK, row_body, 0)

    def group_body(gi, carry):
        for b in range(_NB):
            ci = gi * _NB + b
            wait_in(b)
            compute(b)
            start_out(b, ci)
            b2 = (b + 2) % _NB

            @pl.when(ci >= 2)
            def _():
                wait_out(b2)

            @pl.when(ci + 2 < nchunks)
            def _():
                start_in(b2, ci + 2)

        return carry

    lax.fori_loop(0, nchunks // _NB, group_body, 0)
    wait_out((nchunks - 2) % _NB)
    wait_out((nchunks - 1) % _NB)

    @pl.when(wid == 0)
    def _():
        gbuf[...] = g_sig
        pltpu.sync_copy(gbuf, g_hbm)


def kernel(x, gate, noise, idx, perm):
    b, t, d = x.shape
    rows = b * t
    x2 = x.reshape(rows, d)
    n2 = noise.reshape(rows, _K)

    mesh = plsc.VectorSubcoreMesh(core_axis_name="c", subcore_axis_name="s",
                                  num_cores=_NC, num_subcores=_NS)
    out2, gout = pl.kernel(
        _sc_body,
        out_type=[
            jax.ShapeDtypeStruct((rows, d), jnp.float32),
            jax.ShapeDtypeStruct((_L,), jnp.float32),
        ],
        mesh=mesh,
        compiler_params=pltpu.CompilerParams(needs_layout_passes=False),
        scratch_types=[
            [pltpu.VMEM((_CHUNK, d), jnp.float32) for _ in range(_NB)],
            [pltpu.VMEM((_CHUNK, _K), jnp.float32) for _ in range(_NB)],
            [pltpu.VMEM((_SB, _K), jnp.float32) for _ in range(2)],
            [pltpu.SemaphoreType.DMA for _ in range(2)],
            pltpu.VMEM((_KP,), jnp.int32),
            pltpu.VMEM((_KP,), jnp.int32),
            pltpu.VMEM((_KP,), jnp.float32),
            pltpu.VMEM((_L,), jnp.float32),
            pltpu.VMEM((_KP,), jnp.float32),
            pltpu.VMEM((_NS * _KP,), jnp.float32),
            pltpu.VMEM_SHARED((_NS * _KP,), jnp.float32),
            [pltpu.SemaphoreType.DMA for _ in range(_NB)],
            [pltpu.SemaphoreType.DMA for _ in range(_NB)],
            [pltpu.SemaphoreType.DMA for _ in range(_NB)],
        ],
    )(x2, n2, idx.astype(jnp.int32), perm.astype(jnp.int32),
      gate.reshape(1).astype(jnp.float32))

    return out2.reshape(b, t, d), gout[0]
